# Initial kernel scaffold; baseline (speedup 1.0000x reference)
import jax
import jax.numpy as jnp
from jax.experimental import pallas as pl


def kernel(table, inputs, targets):
    raise NotImplementedError("write your pallas kernel here")


# speedup vs baseline: 1.3905x; 1.3905x over previous
import functools

import jax
import jax.numpy as jnp
from jax import lax
from jax.experimental import pallas as pl
from jax.experimental.pallas import tpu as pltpu, tpu_sc as plsc

_NC = 2
_NS = 16
_L = 16
_NW = _NC * _NS


def _lse_body(x_ref, lse_ref):
    x = x_ref[...]
    m = jnp.max(x, axis=1, keepdims=True)
    s = jnp.sum(jnp.exp(x - m), axis=1, keepdims=True)
    lse_ref[...] = m + jnp.log(s)


def _make_sc_main(V, BT, CH):
    PW = BT // _NW
    NCH = PW // CH
    mesh = plsc.VectorSubcoreMesh(core_axis_name="c", subcore_axis_name="s")

    @functools.partial(
        pl.kernel,
        out_type=(
            jax.ShapeDtypeStruct((BT, V), jnp.float32),
            jax.ShapeDtypeStruct((_NW, _L), jnp.float32),
        ),
        mesh=mesh,
        compiler_params=pltpu.CompilerParams(
            needs_layout_passes=False, use_tc_tiling_on_sc=False
        ),
        scratch_types=[
            pltpu.VMEM((PW,), jnp.int32),
            pltpu.VMEM((CH,), jnp.int32),
            pltpu.VMEM((1024,), jnp.float32),
            pltpu.VMEM((CH, V), jnp.float32),
            pltpu.VMEM((_L,), jnp.float32),
            pltpu.SemaphoreType.DMA,
        ],
    )
    def sc_main(table_hbm, idx_hbm, tgt_hbm, lse_hbm, out_hbm, part_hbm,
                idx_v, tgt_v, lse_v, rows_v, part_v, gsem):
        wid = lax.axis_index("s") * _NC + lax.axis_index("c")
        base = wid * PW
        pltpu.sync_copy(idx_hbm.at[pl.ds(base, PW)], idx_v)
        pltpu.sync_copy(lse_hbm, lse_v)

        def lse_step(j, acc):
            iv = idx_v[pl.ds(j * _L, _L)]
            return acc + plsc.load_gather(lse_v, [iv])

        acc = lax.fori_loop(0, PW // _L, lse_step, jnp.zeros((_L,), jnp.float32))

        def chunk_step(g, acc):
            pltpu.sync_copy(tgt_hbm.at[pl.ds(base + g * CH, CH)], tgt_v)
            pltpu.async_copy(
                table_hbm.at[idx_v.at[pl.ds(g * CH, CH)]], rows_v, gsem
            ).wait()
            for u in range(CH // _L):
                rvec = lax.iota(jnp.int32, _L) + u * _L
                tvec = tgt_v[pl.ds(u * _L, _L)]
                acc = acc - plsc.load_gather(rows_v, [rvec, tvec])
            pltpu.sync_copy(rows_v, out_hbm.at[pl.ds(base + g * CH, CH)])
            return acc

        acc = lax.fori_loop(0, NCH, chunk_step, acc)

        part_v[...] = acc
        pltpu.sync_copy(part_v, part_hbm.at[wid])

    return sc_main


def kernel(table, inputs, targets):
    V = table.shape[0]
    B, T = inputs.shape
    BT = B * T
    idx = inputs.reshape(BT).astype(jnp.int32)
    tgt = targets.reshape(BT).astype(jnp.int32)

    lse = pl.pallas_call(
        _lse_body,
        out_shape=jax.ShapeDtypeStruct((V, 1), jnp.float32),
    )(table)
    lse_pad = jnp.pad(lse.reshape(V), (0, 1024 - V))

    logits_flat, parts = _make_sc_main(V, BT, 64)(table, idx, tgt, lse_pad)
    loss = jnp.sum(parts) / BT
    return logits_flat.reshape(B, T, V), loss


# speedup vs baseline: 1.3927x; 1.0016x over previous
import functools

import jax
import jax.numpy as jnp
from jax import lax
from jax.experimental import pallas as pl
from jax.experimental.pallas import tpu as pltpu, tpu_sc as plsc

_NC = 2
_NS = 16
_L = 16
_NW = _NC * _NS


def _lse_body(x_ref, lse_ref):
    x = x_ref[...]
    m = jnp.max(x, axis=1, keepdims=True)
    s = jnp.sum(jnp.exp(x - m), axis=1, keepdims=True)
    lse_ref[...] = m + jnp.log(s)


def _make_sc_main(V, B, T):
    BPW = B // _NW
    PW = BPW * T
    mesh = plsc.VectorSubcoreMesh(core_axis_name="c", subcore_axis_name="s")

    @functools.partial(
        pl.kernel,
        out_type=(
            jax.ShapeDtypeStruct((B, T, V), jnp.float32),
            jax.ShapeDtypeStruct((_NW, _L), jnp.float32),
        ),
        mesh=mesh,
        compiler_params=pltpu.CompilerParams(
            needs_layout_passes=False, use_tc_tiling_on_sc=False
        ),
        scratch_types=[
            pltpu.VMEM((BPW, T), jnp.int32),
            pltpu.VMEM((PW,), jnp.int32),
            pltpu.VMEM((PW,), jnp.int32),
            pltpu.VMEM((1024,), jnp.float32),
            pltpu.VMEM((T, V), jnp.float32),
            pltpu.VMEM((_L,), jnp.float32),
            pltpu.SemaphoreType.DMA,
        ],
    )
    def sc_main(table_hbm, in2d_hbm, inflat_hbm, tgt_hbm, lse_hbm,
                out_hbm, part_hbm,
                idx2d_v, idx_v, tgt_v, lse_v, rows_v, part_v, gsem):
        wid = lax.axis_index("s") * _NC + lax.axis_index("c")
        b0 = wid * BPW
        pltpu.sync_copy(in2d_hbm.at[pl.ds(b0, BPW)], idx2d_v)
        pltpu.sync_copy(inflat_hbm.at[pl.ds(wid * PW, PW)], idx_v)
        pltpu.sync_copy(tgt_hbm.at[pl.ds(wid * PW, PW)], tgt_v)
        pltpu.sync_copy(lse_hbm, lse_v)

        def lse_step(j, acc):
            iv = idx_v[pl.ds(j * _L, _L)]
            return acc + plsc.load_gather(lse_v, [iv])

        acc = lax.fori_loop(0, PW // _L, lse_step, jnp.zeros((_L,), jnp.float32))

        nvec = (T + _L - 1) // _L

        def chunk_step(g, acc):
            pltpu.async_copy(table_hbm.at[idx2d_v.at[g]], rows_v, gsem).wait()
            for h in range(nvec):
                jflat = g * T + h * _L + lax.iota(jnp.int32, _L)
                m = jflat < (g + 1) * T
                jc = jnp.minimum(jflat, PW - 1)
                tvec = plsc.load_gather(tgt_v, [jc])
                rvec = jnp.minimum(jc - g * T, T - 1)
                pick = plsc.load_gather(rows_v, [rvec, tvec], mask=m)
                acc = acc - jnp.where(m, pick, jnp.zeros((_L,), jnp.float32))
            pltpu.sync_copy(rows_v, out_hbm.at[b0 + g])
            return acc

        acc = lax.fori_loop(0, BPW, chunk_step, acc)

        part_v[...] = acc
        pltpu.sync_copy(part_v, part_hbm.at[wid])

    return sc_main


def kernel(table, inputs, targets):
    V = table.shape[0]
    B, T = inputs.shape
    BT = B * T
    in2d = inputs.astype(jnp.int32)
    inflat = in2d.reshape(BT)
    tgt = targets.reshape(BT).astype(jnp.int32)

    lse = pl.pallas_call(
        _lse_body,
        out_shape=jax.ShapeDtypeStruct((V, 1), jnp.float32),
    )(table)
    lse_pad = jnp.pad(lse.reshape(V), (0, 1024 - V))

    logits, parts = _make_sc_main(V, B, T)(table, in2d, inflat, tgt, lse_pad)
    loss = jnp.sum(parts) / BT
    return logits, loss
